# Initial kernel scaffold; baseline (speedup 1.0000x reference)
import jax
import jax.numpy as jnp
from jax.experimental import pallas as pl


def kernel(x, w1, b1, w2, b2, wf1, bf1, wf2, bf2):
    raise NotImplementedError("write your pallas kernel here")


# speedup vs baseline: 12.7997x; 12.7997x over previous
import numpy as np

import jax
import jax.numpy as jnp
from jax.experimental import pallas as pl
from jax.experimental.pallas import tpu as pltpu

B_TILE = 32
N_CLASSES = 10


def _fused_kernel(x_ref, w1p_ref, b1p_ref, w2p_ref, b2p_ref,
                  wf1p_ref, bf1_ref, wf2_ref, bf2_ref, o_ref):
    bt = x_ref.shape[0]
    xb = x_ref[...].astype(jnp.bfloat16)

    acc = [None, None]
    for ki in range(5):
        lhs = xb[:, ki:ki + 24, :].reshape(bt * 24, 28)
        for h in range(2):
            d = jnp.dot(lhs, w1p_ref[h, ki],
                        preferred_element_type=jnp.float32)
            acc[h] = d if acc[h] is None else acc[h] + d
    z1 = jnp.maximum(acc[0], acc[1]).reshape(bt, 12, 2, 120)
    z1 = jnp.max(z1, axis=2)
    h1 = jnp.maximum(z1 + b1p_ref[...], 0.0).astype(jnp.bfloat16)

    acc2 = [None, None]
    for ki in range(5):
        lhs = h1[:, ki:ki + 8, :].reshape(bt * 8, 120)
        for h in range(2):
            d = jnp.dot(lhs, w2p_ref[h, ki],
                        preferred_element_type=jnp.float32)
            acc2[h] = d if acc2[h] is None else acc2[h] + d
    z2 = jnp.maximum(acc2[0], acc2[1]).reshape(bt, 4, 2, 80)
    z2 = jnp.max(z2, axis=2)
    h2 = jnp.maximum(z2 + b2p_ref[...], 0.0).astype(jnp.bfloat16)

    ha = None
    for i2 in range(4):
        d = jnp.dot(h2[:, i2, :], wf1p_ref[i2],
                    preferred_element_type=jnp.float32)
        ha = d if ha is None else ha + d
    h = jnp.maximum(ha + bf1_ref[...], 0.0).astype(jnp.bfloat16)
    y = jnp.dot(h, wf2_ref[...],
                preferred_element_type=jnp.float32) + bf2_ref[...]

    lane = jax.lax.broadcasted_iota(jnp.int32, (1, 128), 1)
    y = jnp.where(lane < N_CLASSES, y, -1e30)
    mx = jnp.max(y, axis=-1, keepdims=True)
    lse = jnp.log(jnp.sum(jnp.exp(y - mx), axis=-1, keepdims=True)) + mx
    o_ref[...] = y - lse


_E1 = np.zeros((2, 28, 12, 5), np.float32)
for _h in range(2):
    for _j in range(12):
        for _kj in range(5):
            _E1[_h, 2 * _j + _h + _kj, _j, _kj] = 1.0
_E2 = np.zeros((2, 12, 4, 5), np.float32)
for _h in range(2):
    for _j in range(4):
        for _kj in range(5):
            _E2[_h, 2 * _j + _h + _kj, _j, _kj] = 1.0


def _prep(w1, b1, w2, b2, wf1):
    k1 = w1[:25, :10].astype(jnp.float32).reshape(5, 5, 10)
    w1p = jnp.einsum('hNjk,ikc->hiNjc', jnp.asarray(_E1), k1)
    w1p = w1p.reshape(2, 5, 28, 120).astype(jnp.bfloat16)
    b1p = jnp.tile(b1[:, :10], (1, 12))

    k2 = w2.reshape(5, 5, 128, 128)[:, :, :10, :20].astype(jnp.float32)
    w2p = jnp.einsum('hNjk,ikcd->hiNcjd', jnp.asarray(_E2), k2)
    w2p = w2p.reshape(2, 5, 120, 80).astype(jnp.bfloat16)
    b2p = jnp.tile(b2[:, :20], (1, 4))

    wf1p = wf1.reshape(4, 4, 128, 128)[:, :, :20, :].reshape(4, 80, 128)
    return w1p, b1p, w2p, b2p, wf1p


@jax.jit
def kernel(x, w1, b1, w2, b2, wf1, bf1, wf2, bf2):
    B = x.shape[0]
    w1p, b1p, w2p, b2p, wf1p = _prep(w1, b1, w2, b2, wf1)
    xr = x.reshape(B, 28, 28)

    out = pl.pallas_call(
        _fused_kernel,
        out_shape=jax.ShapeDtypeStruct((B, 128), jnp.float32),
        grid=(B // B_TILE,),
        in_specs=[
            pl.BlockSpec((B_TILE, 28, 28), lambda b: (b, 0, 0)),
            pl.BlockSpec((2, 5, 28, 120), lambda b: (0, 0, 0, 0)),
            pl.BlockSpec((1, 120), lambda b: (0, 0)),
            pl.BlockSpec((2, 5, 120, 80), lambda b: (0, 0, 0, 0)),
            pl.BlockSpec((1, 80), lambda b: (0, 0)),
            pl.BlockSpec((4, 80, 128), lambda b: (0, 0, 0)),
            pl.BlockSpec((1, 128), lambda b: (0, 0)),
            pl.BlockSpec((128, 128), lambda b: (0, 0)),
            pl.BlockSpec((1, 128), lambda b: (0, 0)),
        ],
        out_specs=pl.BlockSpec((B_TILE, 128), lambda b: (b, 0)),
        compiler_params=pltpu.CompilerParams(
            dimension_semantics=("parallel",),
            vmem_limit_bytes=64 * 1024 * 1024),
    )(xr, w1p, b1p, w2p, b2p, wf1p, bf1, wf2, bf2)

    return out[:B, :N_CLASSES]
